# baseline (speedup 1.0000x reference)
import functools
import jax
import jax.numpy as jnp
from jax import lax
from jax.experimental import pallas as pl
from jax.experimental.pallas import tpu as pltpu
from jax.experimental.pallas import tpu_sc as plsc

_VOCAB = 1000000
_EMB = 64
_OUT = 2
_L = 200
_B = 4096

_NC = 2
_NS = 16
_NW = _NC * _NS
_NB = _B // _NW
_LN = 16
_C0, _C1 = 96, 104

_FC = 16384
_FG = -(-_VOCAB // _FC)
_FQ = _FC // 4
_VF = _FG * _FQ
_HW = 32
_NSUB = _VF * 4


def _fmt_body(tt_ref, out_ref):
    x = tt_ref[...]
    ii = lax.broadcasted_iota(jnp.int32, (_HW, _HW), 0)
    jj = lax.broadcasted_iota(jnp.int32, (_HW, _HW), 1)
    ident = (ii == jj).astype(jnp.float32)
    quarters = []
    for q in range(4):
        xq = lax.slice(x, (0, q * _FQ), (_EMB, (q + 1) * _FQ))
        xlo = lax.slice(xq, (0, 0), (_HW, _FQ))
        xhi = lax.slice(xq, (_HW, 0), (_EMB, _FQ))
        ylo = lax.dot_general(xlo, ident, (((0,), (0,)), ((), ())),
                              preferred_element_type=jnp.float32)
        yhi = lax.dot_general(xhi, ident, (((0,), (0,)), ((), ())),
                              preferred_element_type=jnp.float32)
        ulo = lax.bitcast_convert_type(
            ylo.astype(jnp.bfloat16), jnp.uint16).astype(jnp.uint32)
        uhi = lax.bitcast_convert_type(
            yhi.astype(jnp.bfloat16), jnp.uint16).astype(jnp.uint32)
        packed = lax.bitwise_or(ulo, lax.shift_left(uhi, jnp.uint32(16)))
        quarters.append(lax.bitcast_convert_type(packed, jnp.float32))
    out_ref[...] = jnp.concatenate(quarters, axis=1)


_TC_FMT = pl.pallas_call(
    _fmt_body,
    grid=(_FG,),
    in_specs=[pl.BlockSpec((_EMB, _FC), lambda g: (0, g))],
    out_specs=pl.BlockSpec((_FQ, 4 * _HW), lambda g: (g, 0)),
    out_shape=jax.ShapeDtypeStruct((_VF, 4 * _HW), jnp.float32),
)


def _make_sc_kernel():
    mesh = plsc.VectorSubcoreMesh(core_axis_name="c", subcore_axis_name="s")

    @functools.partial(
        pl.kernel,
        mesh=mesh,
        compiler_params=pltpu.CompilerParams(use_tc_tiling_on_sc=False),
        out_type=jax.ShapeDtypeStruct((_B * _EMB,), jnp.float32),
        scratch_types=[
            pltpu.VMEM((_NB * _L,), jnp.int32),
            pltpu.VMEM((_L, _HW), jnp.float32),
            pltpu.VMEM((_L, _HW), jnp.float32),
            pltpu.VMEM((_NB * _EMB,), jnp.float32),
            pltpu.SemaphoreType.DMA,
            pltpu.SemaphoreType.DMA,
        ],
    )
    def sc_sum_kernel(xTf_hbm, table_hbm, out_hbm,
                      idx_flat, buf0, buf1, sums_flat, sem0, sem1):
        wid = lax.axis_index("s") * _NC + lax.axis_index("c")
        base = wid * _NB
        ioff = pl.multiple_of(base * _L, 8)
        pltpu.sync_copy(xTf_hbm.at[pl.ds(ioff, _NB * _L)], idx_flat)

        c2 = jnp.full((_LN,), 2, jnp.int32)
        c3 = jnp.full((_LN,), 3, jnp.int32)
        c12 = jnp.full((_LN,), 12, jnp.int32)
        c14 = jnp.full((_LN,), 14, jnp.int32)
        m4095 = jnp.full((_LN,), 4095, jnp.int32)

        def split_idx(k, carry):
            koff = pl.multiple_of(k * _LN, 8)
            v = idx_flat[pl.ds(koff, _LN)]
            s = lax.bitwise_or(
                lax.bitwise_or(
                    lax.shift_left(lax.shift_right_logical(v, c14), c14),
                    lax.shift_left(lax.bitwise_and(v, m4095), c2)),
                lax.bitwise_and(lax.shift_right_logical(v, c12), c3))
            idx_flat[pl.ds(koff, _LN)] = s
            return carry

        lax.fori_loop(0, (_NB * _L) // _LN, split_idx, 0, unroll=8)

        zero = jnp.zeros((_LN,), jnp.float32)
        bufs = (buf0, buf1)
        sems = (sem0, sem1)

        def issue(j, slot):
            joff = pl.multiple_of(j * _L, 8)
            joff2 = pl.multiple_of(joff + _C0, 8)
            pltpu.async_copy(table_hbm.at[idx_flat.at[pl.ds(joff, _C0)]],
                             bufs[slot].at[pl.ds(0, _C0)], sems[slot])
            pltpu.async_copy(table_hbm.at[idx_flat.at[pl.ds(joff2, _C1)]],
                             bufs[slot].at[pl.ds(_C0, _C1)], sems[slot])

        def wait(slot):
            pltpu.make_async_copy(table_hbm.at[pl.ds(0, _L)], bufs[slot],
                                  sems[slot]).wait()

        c16u = jnp.full((_LN,), 16, jnp.uint32)
        mhi = jnp.full((_LN,), 0xFFFF0000, jnp.uint32)

        def reduce_store(slot, j):
            buf = bufs[slot]

            def unpk(v):
                u = lax.bitcast_convert_type(v, jnp.uint32)
                a = lax.bitcast_convert_type(lax.shift_left(u, c16u),
                                             jnp.float32)
                b = lax.bitcast_convert_type(lax.bitwise_and(u, mhi),
                                             jnp.float32)
                return a, b

            def red(l, acc):
                a0, b0 = unpk(buf[l, pl.ds(0, _LN)])
                a1, b1 = unpk(buf[l, pl.ds(_LN, _LN)])
                return (acc[0] + a0, acc[1] + a1,
                        acc[2] + b0, acc[3] + b1)

            acc = lax.fori_loop(0, _L, red, (zero, zero, zero, zero),
                                unroll=8)
            soff = pl.multiple_of(j * _EMB, 8)
            for d in range(4):
                sums_flat[pl.ds(soff + d * _LN, _LN)] = acc[d]

        issue(0, 0)
        issue(1, 1)

        def body(jj, carry):
            j = 2 * jj
            wait(0)
            reduce_store(0, j)
            issue(j + 2, 0)
            wait(1)
            reduce_store(1, j + 1)
            issue(j + 3, 1)
            return carry

        lax.fori_loop(0, _NB // 2 - 1, body, 0)

        wait(0)
        reduce_store(0, _NB - 2)
        wait(1)
        reduce_store(1, _NB - 1)

        ooff = pl.multiple_of(base * _EMB, 8)
        pltpu.sync_copy(sums_flat, out_hbm.at[pl.ds(ooff, _NB * _EMB)])

    return sc_sum_kernel


_SC_SUM = _make_sc_kernel()


def _tc_proj_body(sums_ref, w_ref, b_ref, out_ref):
    p = sums_ref[...] * jnp.float32(1.0 / _L)
    out = lax.dot_general(p, w_ref[...], (((1,), (1,)), ((), ())),
                          preferred_element_type=jnp.float32)
    out_ref[...] = out + b_ref[...]


_TC_PROJ = pl.pallas_call(
    _tc_proj_body,
    out_shape=jax.ShapeDtypeStruct((_B, _OUT), jnp.float32),
)


@jax.jit
def kernel(x, table, W, b):
    xTf = jnp.asarray(x, jnp.int32).T.reshape(-1)
    packed = _TC_FMT(table.T)
    sub32 = packed.reshape(_NSUB, _HW)
    sums = _SC_SUM(xTf, sub32).reshape(_B, _EMB)
    return _TC_PROJ(sums, W.astype(jnp.float32),
                    b.astype(jnp.float32)[None, :])


# speedup vs baseline: 1.2970x; 1.2970x over previous
import functools
import jax
import jax.numpy as jnp
from jax import lax
from jax.experimental import pallas as pl
from jax.experimental.pallas import tpu as pltpu
from jax.experimental.pallas import tpu_sc as plsc

_VOCAB = 1000000
_EMB = 64
_FW = 2 * _EMB
_OUT = 2
_L = 200
_B = 4096

_NC = 2
_NS = 16
_NW = _NC * _NS
_NB = _B // _NW
_LN = 16
_C0, _C1 = 96, 104

_FC = 16384
_FG = -(-_VOCAB // _FC)
_FR = _FC // 2
_VF = _FG * _FR
_HW = 32
_NSUB = _VF * 4


def _fmt_body(tt_ref, out_ref):
    x = tt_ref[...]
    ii = lax.broadcasted_iota(jnp.int32, (_EMB, _EMB), 0)
    jj = lax.broadcasted_iota(jnp.int32, (_EMB, _EMB), 1)
    ident = (ii == jj).astype(jnp.float32)
    xl = lax.slice(x, (0, 0), (_EMB, _FR))
    xr = lax.slice(x, (0, _FR), (_EMB, _FC))
    yl = lax.dot_general(xl, ident, (((0,), (0,)), ((), ())),
                         preferred_element_type=jnp.float32)
    yr = lax.dot_general(xr, ident, (((0,), (0,)), ((), ())),
                         preferred_element_type=jnp.float32)
    out_ref[...] = jnp.concatenate([yl, yr], axis=1)


_TC_FMT = pl.pallas_call(
    _fmt_body,
    grid=(_FG,),
    in_specs=[pl.BlockSpec((_EMB, _FC), lambda g: (0, g))],
    out_specs=pl.BlockSpec((_FR, _FW), lambda g: (g, 0)),
    out_shape=jax.ShapeDtypeStruct((_VF, _FW), jnp.float32),
)


def _make_sc_kernel():
    mesh = plsc.VectorSubcoreMesh(core_axis_name="c", subcore_axis_name="s")

    @functools.partial(
        pl.kernel,
        mesh=mesh,
        compiler_params=pltpu.CompilerParams(use_tc_tiling_on_sc=False),
        out_type=jax.ShapeDtypeStruct((_B * _EMB,), jnp.float32),
        scratch_types=[
            pltpu.VMEM((_NB * _L,), jnp.int32),
            pltpu.VMEM((_NB * _L,), jnp.int32),
            pltpu.VMEM((_L, _HW), jnp.float32),
            pltpu.VMEM((_L, _HW), jnp.float32),
            pltpu.VMEM((_L, _HW), jnp.float32),
            pltpu.VMEM((_L, _HW), jnp.float32),
            pltpu.VMEM((_NB * _EMB,), jnp.float32),
            pltpu.SemaphoreType.DMA,
            pltpu.SemaphoreType.DMA,
        ],
    )
    def sc_sum_kernel(xTf_hbm, table_hbm, out_hbm,
                      idxE, idxO, buf0e, buf0o, buf1e, buf1o,
                      sums_flat, sem0, sem1):
        wid = lax.axis_index("s") * _NC + lax.axis_index("c")
        base = wid * _NB
        ioff = pl.multiple_of(base * _L, 8)
        pltpu.sync_copy(xTf_hbm.at[pl.ds(ioff, _NB * _L)], idxE)

        one = jnp.full((_LN,), 1, jnp.int32)
        c2 = jnp.full((_LN,), 2, jnp.int32)
        c13 = jnp.full((_LN,), 13, jnp.int32)
        c14 = jnp.full((_LN,), 14, jnp.int32)
        c15 = jnp.full((_LN,), 15, jnp.int32)
        m8191 = jnp.full((_LN,), 8191, jnp.int32)

        def split_idx(k, carry):
            koff = pl.multiple_of(k * _LN, 8)
            v = idxE[pl.ds(koff, _LN)]
            sA = lax.bitwise_or(
                lax.bitwise_or(
                    lax.shift_left(lax.shift_right_logical(v, c14), c15),
                    lax.shift_left(lax.bitwise_and(v, m8191), c2)),
                lax.shift_left(
                    lax.bitwise_and(lax.shift_right_logical(v, c13), one),
                    one))
            idxE[pl.ds(koff, _LN)] = sA
            idxO[pl.ds(koff, _LN)] = sA + one
            return carry

        lax.fori_loop(0, (_NB * _L) // _LN, split_idx, 0, unroll=8)

        zero = jnp.zeros((_LN,), jnp.float32)

        def issue(j, bufe, bufo, sem):
            joff = pl.multiple_of(j * _L, 8)
            joff2 = pl.multiple_of(joff + _C0, 8)
            pltpu.async_copy(table_hbm.at[idxE.at[pl.ds(joff, _C0)]],
                             bufe.at[pl.ds(0, _C0)], sem)
            pltpu.async_copy(table_hbm.at[idxE.at[pl.ds(joff2, _C1)]],
                             bufe.at[pl.ds(_C0, _C1)], sem)
            pltpu.async_copy(table_hbm.at[idxO.at[pl.ds(joff, _C0)]],
                             bufo.at[pl.ds(0, _C0)], sem)
            pltpu.async_copy(table_hbm.at[idxO.at[pl.ds(joff2, _C1)]],
                             bufo.at[pl.ds(_C0, _C1)], sem)

        def wait(bufe, bufo, sem):
            pltpu.make_async_copy(table_hbm.at[pl.ds(0, _L)], bufe, sem).wait()
            pltpu.make_async_copy(table_hbm.at[pl.ds(0, _L)], bufo, sem).wait()

        def reduce_store(bufe, bufo, j):
            def red(l, acc):
                return (acc[0] + bufe[l, pl.ds(0, _LN)],
                        acc[1] + bufe[l, pl.ds(_LN, _LN)],
                        acc[2] + bufo[l, pl.ds(0, _LN)],
                        acc[3] + bufo[l, pl.ds(_LN, _LN)])

            acc = lax.fori_loop(0, _L, red, (zero, zero, zero, zero),
                                unroll=8)
            soff = pl.multiple_of(j * _EMB, 8)
            for d in range(4):
                sums_flat[pl.ds(soff + d * _LN, _LN)] = acc[d]

        issue(0, buf0e, buf0o, sem0)
        issue(1, buf1e, buf1o, sem1)

        def body(jj, carry):
            j = 2 * jj
            wait(buf0e, buf0o, sem0)
            reduce_store(buf0e, buf0o, j)
            issue(j + 2, buf0e, buf0o, sem0)
            wait(buf1e, buf1o, sem1)
            reduce_store(buf1e, buf1o, j + 1)
            issue(j + 3, buf1e, buf1o, sem1)
            return carry

        lax.fori_loop(0, _NB // 2 - 1, body, 0)

        wait(buf0e, buf0o, sem0)
        reduce_store(buf0e, buf0o, _NB - 2)
        wait(buf1e, buf1o, sem1)
        reduce_store(buf1e, buf1o, _NB - 1)

        ooff = pl.multiple_of(base * _EMB, 8)
        pltpu.sync_copy(sums_flat, out_hbm.at[pl.ds(ooff, _NB * _EMB)])

    return sc_sum_kernel


_SC_SUM = _make_sc_kernel()


def _tc_proj_body(sums_ref, w_ref, b_ref, out_ref):
    p = sums_ref[...] * jnp.float32(1.0 / _L)
    out = lax.dot_general(p, w_ref[...], (((1,), (1,)), ((), ())),
                          preferred_element_type=jnp.float32)
    out_ref[...] = out + b_ref[...]


_TC_PROJ = pl.pallas_call(
    _tc_proj_body,
    out_shape=jax.ShapeDtypeStruct((_B, _OUT), jnp.float32),
)


@jax.jit
def kernel(x, table, W, b):
    xTf = jnp.asarray(x, jnp.int32).T.reshape(-1)
    fused = _TC_FMT(table.T)
    sub32 = fused.reshape(_NSUB, _HW)
    sums = _SC_SUM(xTf, sub32).reshape(_B, _EMB)
    return _TC_PROJ(sums, W.astype(jnp.float32),
                    b.astype(jnp.float32)[None, :])


# speedup vs baseline: 1.3474x; 1.0389x over previous
import functools
import jax
import jax.numpy as jnp
from jax import lax
from jax.experimental import pallas as pl
from jax.experimental.pallas import tpu as pltpu
from jax.experimental.pallas import tpu_sc as plsc

_VOCAB = 1000000
_EMB = 64
_FW = 2 * _EMB
_OUT = 2
_L = 200
_B = 4096

_NC = 2
_NS = 16
_NW = _NC * _NS
_NB = _B // _NW
_LN = 16
_C0, _C1 = 96, 104

_FC = 32768
_FG = -(-_VOCAB // _FC)
_FR = _FC // 2
_VF = _FG * _FR
_HW = 32
_NSUB = _VF * 4


def _fmt_body(tt_ref, out_ref):
    x = tt_ref[...]
    ii = lax.broadcasted_iota(jnp.int32, (_EMB, _EMB), 0)
    jj = lax.broadcasted_iota(jnp.int32, (_EMB, _EMB), 1)
    ident = (ii == jj).astype(jnp.float32)
    xl = lax.slice(x, (0, 0), (_EMB, _FR))
    xr = lax.slice(x, (0, _FR), (_EMB, _FC))
    yl = lax.dot_general(xl, ident, (((0,), (0,)), ((), ())),
                         preferred_element_type=jnp.float32)
    yr = lax.dot_general(xr, ident, (((0,), (0,)), ((), ())),
                         preferred_element_type=jnp.float32)
    out_ref[...] = jnp.concatenate([yl, yr], axis=1)


_TC_FMT = pl.pallas_call(
    _fmt_body,
    grid=(_FG,),
    in_specs=[pl.BlockSpec((_EMB, _FC), lambda g: (0, g))],
    out_specs=pl.BlockSpec((_FR, _FW), lambda g: (g, 0)),
    out_shape=jax.ShapeDtypeStruct((_VF, _FW), jnp.float32),
)


def _make_sc_kernel():
    mesh = plsc.VectorSubcoreMesh(core_axis_name="c", subcore_axis_name="s")

    @functools.partial(
        pl.kernel,
        mesh=mesh,
        compiler_params=pltpu.CompilerParams(use_tc_tiling_on_sc=False),
        out_type=jax.ShapeDtypeStruct((_B * _EMB,), jnp.float32),
        scratch_types=[
            pltpu.VMEM((_NB * _L,), jnp.int32),
            pltpu.VMEM((_NB * _L,), jnp.int32),
            pltpu.VMEM((_L, _HW), jnp.float32),
            pltpu.VMEM((_L, _HW), jnp.float32),
            pltpu.VMEM((_L, _HW), jnp.float32),
            pltpu.VMEM((_L, _HW), jnp.float32),
            pltpu.VMEM((_NB * _EMB,), jnp.float32),
            pltpu.SemaphoreType.DMA,
            pltpu.SemaphoreType.DMA,
        ],
    )
    def sc_sum_kernel(xTf_hbm, table_hbm, out_hbm,
                      idxE, idxO, buf0e, buf0o, buf1e, buf1o,
                      sums_flat, sem0, sem1):
        wid = lax.axis_index("s") * _NC + lax.axis_index("c")
        base = wid * _NB
        ioff = pl.multiple_of(base * _L, 8)
        pltpu.sync_copy(xTf_hbm.at[pl.ds(ioff, _NB * _L)], idxE)

        one = jnp.full((_LN,), 1, jnp.int32)
        c2 = jnp.full((_LN,), 2, jnp.int32)
        c14 = jnp.full((_LN,), 14, jnp.int32)
        c15 = jnp.full((_LN,), 15, jnp.int32)
        c16 = jnp.full((_LN,), 16, jnp.int32)
        m16383 = jnp.full((_LN,), 16383, jnp.int32)

        def split_idx(k, carry):
            koff = pl.multiple_of(k * _LN, 8)
            v = idxE[pl.ds(koff, _LN)]
            sA = lax.bitwise_or(
                lax.bitwise_or(
                    lax.shift_left(lax.shift_right_logical(v, c15), c16),
                    lax.shift_left(lax.bitwise_and(v, m16383), c2)),
                lax.shift_left(
                    lax.bitwise_and(lax.shift_right_logical(v, c14), one),
                    one))
            idxE[pl.ds(koff, _LN)] = sA
            idxO[pl.ds(koff, _LN)] = sA + one
            return carry

        lax.fori_loop(0, (_NB * _L) // _LN, split_idx, 0, unroll=8)

        zero = jnp.zeros((_LN,), jnp.float32)

        def issue(j, bufe, bufo, sem):
            joff = pl.multiple_of(j * _L, 8)
            joff2 = pl.multiple_of(joff + _C0, 8)
            pltpu.async_copy(table_hbm.at[idxE.at[pl.ds(joff, _C0)]],
                             bufe.at[pl.ds(0, _C0)], sem)
            pltpu.async_copy(table_hbm.at[idxE.at[pl.ds(joff2, _C1)]],
                             bufe.at[pl.ds(_C0, _C1)], sem)
            pltpu.async_copy(table_hbm.at[idxO.at[pl.ds(joff, _C0)]],
                             bufo.at[pl.ds(0, _C0)], sem)
            pltpu.async_copy(table_hbm.at[idxO.at[pl.ds(joff2, _C1)]],
                             bufo.at[pl.ds(_C0, _C1)], sem)

        def wait(bufe, bufo, sem):
            pltpu.make_async_copy(table_hbm.at[pl.ds(0, _L)], bufe, sem).wait()
            pltpu.make_async_copy(table_hbm.at[pl.ds(0, _L)], bufo, sem).wait()

        def reduce_store(bufe, bufo, j):
            def red(l, acc):
                return (acc[0] + bufe[l, pl.ds(0, _LN)],
                        acc[1] + bufe[l, pl.ds(_LN, _LN)],
                        acc[2] + bufo[l, pl.ds(0, _LN)],
                        acc[3] + bufo[l, pl.ds(_LN, _LN)])

            acc = lax.fori_loop(0, _L, red, (zero, zero, zero, zero),
                                unroll=8)
            soff = pl.multiple_of(j * _EMB, 8)
            for d in range(4):
                sums_flat[pl.ds(soff + d * _LN, _LN)] = acc[d]

        issue(0, buf0e, buf0o, sem0)
        issue(1, buf1e, buf1o, sem1)

        def body(jj, carry):
            j = 2 * jj
            wait(buf0e, buf0o, sem0)
            reduce_store(buf0e, buf0o, j)
            issue(j + 2, buf0e, buf0o, sem0)
            wait(buf1e, buf1o, sem1)
            reduce_store(buf1e, buf1o, j + 1)
            issue(j + 3, buf1e, buf1o, sem1)
            return carry

        lax.fori_loop(0, _NB // 2 - 1, body, 0)

        wait(buf0e, buf0o, sem0)
        reduce_store(buf0e, buf0o, _NB - 2)
        wait(buf1e, buf1o, sem1)
        reduce_store(buf1e, buf1o, _NB - 1)

        ooff = pl.multiple_of(base * _EMB, 8)
        pltpu.sync_copy(sums_flat, out_hbm.at[pl.ds(ooff, _NB * _EMB)])

    return sc_sum_kernel


_SC_SUM = _make_sc_kernel()


def _tc_proj_body(sums_ref, w_ref, b_ref, out_ref):
    p = sums_ref[...] * jnp.float32(1.0 / _L)
    out = lax.dot_general(p, w_ref[...], (((1,), (1,)), ((), ())),
                          preferred_element_type=jnp.float32)
    out_ref[...] = out + b_ref[...]


_TC_PROJ = pl.pallas_call(
    _tc_proj_body,
    out_shape=jax.ShapeDtypeStruct((_B, _OUT), jnp.float32),
)


@jax.jit
def kernel(x, table, W, b):
    xTf = jnp.asarray(x, jnp.int32).T.reshape(-1)
    fused = _TC_FMT(table.T)
    sub32 = fused.reshape(_NSUB, _HW)
    sums = _SC_SUM(xTf, sub32).reshape(_B, _EMB)
    return _TC_PROJ(sums, W.astype(jnp.float32),
                    b.astype(jnp.float32)[None, :])
